# baseline (speedup 1.0000x reference)
import functools

import jax
import jax.numpy as jnp
from jax import lax
from jax.experimental import pallas as pl
from jax.experimental.pallas import tpu as pltpu
from jax.experimental.pallas import tpu_sc as plsc

CH = 512


def _gather_sc(idx_flat, table):
    B = idx_flat.shape[0]
    D = table.shape[1]
    info = plsc.get_sparse_core_info()
    NC, NS = info.num_cores, info.num_subcores
    NW = NC * NS
    assert B % (NW * CH) == 0
    b_per_w = B // NW
    n_chunks = b_per_w // CH

    mesh = plsc.VectorSubcoreMesh(core_axis_name="c", subcore_axis_name="s")

    scratch = (
        [pltpu.VMEM((CH,), jnp.int32) for _ in range(2)]
        + [pltpu.VMEM((CH, D), jnp.float32) for _ in range(2)]
        + [pltpu.SemaphoreType.DMA for _ in range(2)]
    )

    @functools.partial(
        pl.kernel,
        mesh=mesh,
        out_type=jax.ShapeDtypeStruct((B, D), jnp.float32),
        scratch_types=scratch,
        compiler_params=pltpu.CompilerParams(
            use_tc_tiling_on_sc=False, needs_layout_passes=False
        ),
    )
    def k(idx_hbm, table_hbm, out_hbm, idx0, idx1, rows0, rows1, sem0, sem1):
        idx_v = (idx0, idx1)
        rows_v = (rows0, rows1)
        sem = (sem0, sem1)

        wid = lax.axis_index("s") * NC + lax.axis_index("c")
        base = wid * b_per_w

        def stage_and_gather(c, p):
            pltpu.sync_copy(idx_hbm.at[pl.ds(base + c * CH, CH)], idx_v[p])
            pltpu.make_async_copy(
                table_hbm.at[idx_v[p]], rows_v[p], sem[p]
            ).start()

        def drain(c, p):
            pltpu.make_async_copy(
                table_hbm.at[idx_v[p]], rows_v[p], sem[p]
            ).wait()
            pltpu.sync_copy(
                rows_v[p], out_hbm.at[pl.ds(base + c * CH, CH)]
            )

        assert n_chunks % 2 == 0
        stage_and_gather(0, 0)

        def pair(i, _):
            c = 2 * i
            stage_and_gather(c + 1, 1)
            drain(c, 0)
            stage_and_gather(c + 2, 0)
            drain(c + 1, 1)
            return 0

        lax.fori_loop(0, n_chunks // 2 - 1, pair, 0)
        stage_and_gather(n_chunks - 1, 1)
        drain(n_chunks - 2, 0)
        drain(n_chunks - 1, 1)

    return k(idx_flat, table)


def _mask_tc(term):
    def mk(t_ref, o_ref):
        o_ref[...] = t_ref[...] == 0

    return pl.pallas_call(
        mk,
        out_shape=jax.ShapeDtypeStruct(term.shape, jnp.bool_),
    )(term)


@jax.jit
def kernel(term, table):
    bsz, hist = term.shape
    D = table.shape[1]
    idx_flat = term.reshape(bsz * hist)
    rows = _gather_sc(idx_flat, table)
    emb = rows.reshape(bsz, hist, D)
    mask = _mask_tc(term)
    return emb, mask


# speedup vs baseline: 1.0121x; 1.0121x over previous
import functools

import jax
import jax.numpy as jnp
from jax import lax
from jax.experimental import pallas as pl
from jax.experimental.pallas import tpu as pltpu
from jax.experimental.pallas import tpu_sc as plsc

CH = 1280


def _gather_sc(idx_flat, table):
    B = idx_flat.shape[0]
    D = table.shape[1]
    info = plsc.get_sparse_core_info()
    NC, NS = info.num_cores, info.num_subcores
    NW = NC * NS
    assert B % (NW * CH) == 0
    b_per_w = B // NW
    n_chunks = b_per_w // CH

    mesh = plsc.VectorSubcoreMesh(core_axis_name="c", subcore_axis_name="s")

    scratch = (
        [pltpu.VMEM((CH,), jnp.int32) for _ in range(2)]
        + [pltpu.VMEM((CH, D), jnp.float32) for _ in range(2)]
        + [pltpu.SemaphoreType.DMA for _ in range(2)]
    )

    @functools.partial(
        pl.kernel,
        mesh=mesh,
        out_type=jax.ShapeDtypeStruct((B, D), jnp.float32),
        scratch_types=scratch,
        compiler_params=pltpu.CompilerParams(
            use_tc_tiling_on_sc=False, needs_layout_passes=False
        ),
    )
    def k(idx_hbm, table_hbm, out_hbm, idx0, idx1, rows0, rows1, sem0, sem1):
        idx_v = (idx0, idx1)
        rows_v = (rows0, rows1)
        sem = (sem0, sem1)

        wid = lax.axis_index("s") * NC + lax.axis_index("c")
        base = wid * b_per_w

        def stage_and_gather(c, p):
            pltpu.sync_copy(idx_hbm.at[pl.ds(base + c * CH, CH)], idx_v[p])
            pltpu.make_async_copy(
                table_hbm.at[idx_v[p]], rows_v[p], sem[p]
            ).start()

        def drain(c, p):
            pltpu.make_async_copy(
                table_hbm.at[idx_v[p]], rows_v[p], sem[p]
            ).wait()
            pltpu.sync_copy(
                rows_v[p], out_hbm.at[pl.ds(base + c * CH, CH)]
            )

        assert n_chunks % 2 == 0
        stage_and_gather(0, 0)

        def pair(i, _):
            c = 2 * i
            stage_and_gather(c + 1, 1)
            drain(c, 0)
            stage_and_gather(c + 2, 0)
            drain(c + 1, 1)
            return 0

        lax.fori_loop(0, n_chunks // 2 - 1, pair, 0)
        stage_and_gather(n_chunks - 1, 1)
        drain(n_chunks - 2, 0)
        drain(n_chunks - 1, 1)

    return k(idx_flat, table)


def _mask_tc(term):
    def mk(t_ref, o_ref):
        o_ref[...] = t_ref[...] == 0

    return pl.pallas_call(
        mk,
        out_shape=jax.ShapeDtypeStruct(term.shape, jnp.bool_),
    )(term)


@jax.jit
def kernel(term, table):
    bsz, hist = term.shape
    D = table.shape[1]
    idx_flat = term.reshape(bsz * hist)
    rows = _gather_sc(idx_flat, table)
    emb = rows.reshape(bsz, hist, D)
    mask = _mask_tc(term)
    return emb, mask


# speedup vs baseline: 1.0147x; 1.0025x over previous
import functools

import jax
import jax.numpy as jnp
from jax import lax
from jax.experimental import pallas as pl
from jax.experimental.pallas import tpu as pltpu
from jax.experimental.pallas import tpu_sc as plsc

CH = 1600


def _gather_sc(idx_flat, table):
    B = idx_flat.shape[0]
    D = table.shape[1]
    info = plsc.get_sparse_core_info()
    NC, NS = info.num_cores, info.num_subcores
    NW = NC * NS
    assert B % (NW * CH) == 0
    b_per_w = B // NW
    n_chunks = b_per_w // CH

    mesh = plsc.VectorSubcoreMesh(core_axis_name="c", subcore_axis_name="s")

    scratch = (
        [pltpu.VMEM((CH,), jnp.int32) for _ in range(2)]
        + [pltpu.VMEM((CH, D), jnp.float32) for _ in range(2)]
        + [pltpu.SemaphoreType.DMA for _ in range(2)]
    )

    @functools.partial(
        pl.kernel,
        mesh=mesh,
        out_type=jax.ShapeDtypeStruct((B, D), jnp.float32),
        scratch_types=scratch,
        compiler_params=pltpu.CompilerParams(
            use_tc_tiling_on_sc=False, needs_layout_passes=False
        ),
    )
    def k(idx_hbm, table_hbm, out_hbm, idx0, idx1, rows0, rows1, sem0, sem1):
        idx_v = (idx0, idx1)
        rows_v = (rows0, rows1)
        sem = (sem0, sem1)

        wid = lax.axis_index("s") * NC + lax.axis_index("c")
        base = wid * b_per_w

        def stage_and_gather(c, p):
            pltpu.sync_copy(idx_hbm.at[pl.ds(base + c * CH, CH)], idx_v[p])
            pltpu.make_async_copy(
                table_hbm.at[idx_v[p]], rows_v[p], sem[p]
            ).start()

        def drain(c, p):
            pltpu.make_async_copy(
                table_hbm.at[idx_v[p]], rows_v[p], sem[p]
            ).wait()
            pltpu.sync_copy(
                rows_v[p], out_hbm.at[pl.ds(base + c * CH, CH)]
            )

        assert n_chunks % 2 == 0
        stage_and_gather(0, 0)

        def pair(i, _):
            c = 2 * i
            stage_and_gather(c + 1, 1)
            drain(c, 0)
            stage_and_gather(c + 2, 0)
            drain(c + 1, 1)
            return 0

        lax.fori_loop(0, n_chunks // 2 - 1, pair, 0)
        stage_and_gather(n_chunks - 1, 1)
        drain(n_chunks - 2, 0)
        drain(n_chunks - 1, 1)

    return k(idx_flat, table)


def _mask_tc(term):
    def mk(t_ref, o_ref):
        o_ref[...] = t_ref[...] == 0

    return pl.pallas_call(
        mk,
        out_shape=jax.ShapeDtypeStruct(term.shape, jnp.bool_),
    )(term)


@jax.jit
def kernel(term, table):
    bsz, hist = term.shape
    D = table.shape[1]
    idx_flat = term.reshape(bsz * hist)
    rows = _gather_sc(idx_flat, table)
    emb = rows.reshape(bsz, hist, D)
    mask = _mask_tc(term)
    return emb, mask
